# baseline (speedup 1.0000x reference)
import jax
import jax.numpy as jnp
from jax.experimental import pallas as pl
from jax.experimental.pallas import tpu as pltpu


def _make_lstm_body(seq_len, bh, d_pad, hid):
    def body(x_ref, wih_ref, whh_ref, b_ref,
             hid_ref, cell_ref,
             seq_ref, gate_ref):
        wih = wih_ref[0]
        whh = whh_ref[0]
        b = b_ref[0]

        @pl.when(pl.program_id(1) == 0)
        def _():
            x2d = x_ref[...].reshape(seq_len * bh, d_pad)
            gate_ref[...] = (
                jnp.dot(x2d, wih, preferred_element_type=jnp.float32) + b)

        @pl.when(pl.program_id(1) > 0)
        def _():
            gate_ref[...] = (
                jnp.dot(seq_ref[...], wih[:hid, :],
                        preferred_element_type=jnp.float32) + b)

        h = jnp.zeros((bh, hid), jnp.float32)
        c = jnp.zeros((bh, hid), jnp.float32)
        hb = None
        for t in range(seq_len):
            g_in = gate_ref[t * bh:(t + 1) * bh, :]
            if t == 0:
                gates = g_in
            else:
                gates = jnp.dot(hb, whh, preferred_element_type=jnp.float32) + g_in

            i_g = jax.nn.sigmoid(gates[:, 0 * hid:1 * hid])
            f_g = jax.nn.sigmoid(gates[:, 1 * hid:2 * hid])
            g_g = jnp.tanh(gates[:, 2 * hid:3 * hid])
            o_g = jax.nn.sigmoid(gates[:, 3 * hid:4 * hid])

            c = f_g * c + i_g * g_g
            h = o_g * jnp.tanh(c)
            hb = h.astype(jnp.bfloat16)
            seq_ref[t * bh:(t + 1) * bh, :] = hb

        hid_ref[0] = h
        cell_ref[0] = c

    return body


def kernel(x, w_ih_all, w_hh_all, b_all):
    num_layers, d_pad, four_h = w_ih_all.shape
    hid = four_h // 4
    B, T, D = x.shape

    n_cores = 2
    bh = max(8, -(-B // (8 * n_cores)) * 8)
    b_pad = n_cores * bh

    xk = jnp.transpose(x, (1, 0, 2))
    xk = jnp.pad(xk, ((0, 0), (0, b_pad - B), (0, d_pad - D)))
    xk = xk.astype(jnp.bfloat16)

    wih = w_ih_all.astype(jnp.bfloat16)
    whh = w_hh_all.astype(jnp.bfloat16)

    body = _make_lstm_body(T, bh, d_pad, hid)

    hidden, cell = pl.pallas_call(
        body,
        grid=(n_cores, num_layers),
        in_specs=[
            pl.BlockSpec((T, bh, d_pad), lambda c, l: (0, c, 0)),
            pl.BlockSpec((1, d_pad, four_h), lambda c, l: (l, 0, 0)),
            pl.BlockSpec((1, hid, four_h), lambda c, l: (l, 0, 0)),
            pl.BlockSpec((1, 1, four_h), lambda c, l: (l, 0, 0)),
        ],
        out_specs=[
            pl.BlockSpec((1, bh, hid), lambda c, l: (l, c, 0)),
            pl.BlockSpec((1, bh, hid), lambda c, l: (l, c, 0)),
        ],
        out_shape=(
            jax.ShapeDtypeStruct((num_layers, b_pad, hid), jnp.float32),
            jax.ShapeDtypeStruct((num_layers, b_pad, hid), jnp.float32),
        ),
        scratch_shapes=[
            pltpu.VMEM((T * bh, hid), jnp.bfloat16),
            pltpu.VMEM((T * bh, four_h), jnp.float32),
        ],
        compiler_params=pltpu.CompilerParams(
            dimension_semantics=("parallel", "arbitrary")),
    )(xk, wih, whh, b_all)

    return hidden[:, :B, :], cell[:, :B, :]


# speedup vs baseline: 1.0433x; 1.0433x over previous
import jax
import jax.numpy as jnp
from jax.experimental import pallas as pl
from jax.experimental.pallas import tpu as pltpu


def _make_lstm_body(seq_len, bh, d_pad, hid, n_streams):
    bs = bh // n_streams

    def body(x_ref, wih_ref, whh_ref, b_ref,
             hid_ref, cell_ref,
             seq_ref, gate_ref):
        wih = wih_ref[0]
        whh = whh_ref[0]
        b = b_ref[0]

        @pl.when(pl.program_id(1) == 0)
        def _():
            x2d = x_ref[...].reshape(seq_len * bh, d_pad)
            gate_ref[...] = (
                jnp.dot(x2d, wih, preferred_element_type=jnp.float32) + b)

        @pl.when(pl.program_id(1) > 0)
        def _():
            gate_ref[...] = (
                jnp.dot(seq_ref[...], wih[:hid, :],
                        preferred_element_type=jnp.float32) + b)

        h = [jnp.zeros((bs, hid), jnp.float32) for _ in range(n_streams)]
        c = [jnp.zeros((bs, hid), jnp.float32) for _ in range(n_streams)]
        hb = [None] * n_streams
        for t in range(seq_len):
            for s in range(n_streams):
                r0 = t * bh + s * bs
                g_in = gate_ref[r0:r0 + bs, :]
                if t == 0:
                    gates = g_in
                else:
                    gates = jnp.dot(hb[s], whh,
                                    preferred_element_type=jnp.float32) + g_in

                i_g = jax.nn.sigmoid(gates[:, 0 * hid:1 * hid])
                f_g = jax.nn.sigmoid(gates[:, 1 * hid:2 * hid])
                g_g = jnp.tanh(gates[:, 2 * hid:3 * hid])
                o_g = jax.nn.sigmoid(gates[:, 3 * hid:4 * hid])

                c[s] = f_g * c[s] + i_g * g_g
                h[s] = o_g * jnp.tanh(c[s])
                hb[s] = h[s].astype(jnp.bfloat16)
                seq_ref[r0:r0 + bs, :] = hb[s]

        hid_ref[0] = jnp.concatenate(h, axis=0)
        cell_ref[0] = jnp.concatenate(c, axis=0)

    return body


def kernel(x, w_ih_all, w_hh_all, b_all):
    num_layers, d_pad, four_h = w_ih_all.shape
    hid = four_h // 4
    B, T, D = x.shape

    n_cores = 2
    n_streams = 2
    q = 8 * n_streams
    bh = max(q, -(-B // (q * n_cores)) * q)
    b_pad = n_cores * bh

    xk = jnp.transpose(x, (1, 0, 2))
    xk = jnp.pad(xk, ((0, 0), (0, b_pad - B), (0, d_pad - D)))
    xk = xk.astype(jnp.bfloat16)

    wih = w_ih_all.astype(jnp.bfloat16)
    whh = w_hh_all.astype(jnp.bfloat16)

    body = _make_lstm_body(T, bh, d_pad, hid, n_streams)

    hidden, cell = pl.pallas_call(
        body,
        grid=(n_cores, num_layers),
        in_specs=[
            pl.BlockSpec((T, bh, d_pad), lambda c, l: (0, c, 0)),
            pl.BlockSpec((1, d_pad, four_h), lambda c, l: (l, 0, 0)),
            pl.BlockSpec((1, hid, four_h), lambda c, l: (l, 0, 0)),
            pl.BlockSpec((1, 1, four_h), lambda c, l: (l, 0, 0)),
        ],
        out_specs=[
            pl.BlockSpec((1, bh, hid), lambda c, l: (l, c, 0)),
            pl.BlockSpec((1, bh, hid), lambda c, l: (l, c, 0)),
        ],
        out_shape=(
            jax.ShapeDtypeStruct((num_layers, b_pad, hid), jnp.float32),
            jax.ShapeDtypeStruct((num_layers, b_pad, hid), jnp.float32),
        ),
        scratch_shapes=[
            pltpu.VMEM((T * bh, hid), jnp.bfloat16),
            pltpu.VMEM((T * bh, four_h), jnp.float32),
        ],
        compiler_params=pltpu.CompilerParams(
            dimension_semantics=("parallel", "arbitrary")),
    )(xk, wih, whh, b_all)

    return hidden[:, :B, :], cell[:, :B, :]


# speedup vs baseline: 1.5482x; 1.4839x over previous
import jax
import jax.numpy as jnp
from jax.experimental import pallas as pl
from jax.experimental.pallas import tpu as pltpu


def _make_lstm_body(seq_len, b_pad, d_pad, hid, n_streams):
    bs = b_pad // n_streams
    four_h = 4 * hid

    def body(x_ref, wih_ref, whh_ref, b_ref,
             hid_ref, cell_ref,
             xt_ref, seq_ref, gate_ref):
        wih = wih_ref[0].astype(jnp.bfloat16)
        whh = whh_ref[0].astype(jnp.bfloat16)
        b = b_ref[0]

        @pl.when(pl.program_id(0) == 0)
        def _():
            for t in range(seq_len):
                xt_ref[t * b_pad:(t + 1) * b_pad, :] = (
                    x_ref[:, t, :].astype(jnp.bfloat16))
            gate_ref[...] = (
                jnp.dot(xt_ref[...], wih, preferred_element_type=jnp.float32) + b)

        @pl.when(pl.program_id(0) > 0)
        def _():
            gate_ref[...] = (
                jnp.dot(seq_ref[...], wih[:hid, :],
                        preferred_element_type=jnp.float32) + b)

        h = [jnp.zeros((bs, hid), jnp.float32) for _ in range(n_streams)]
        c = [jnp.zeros((bs, hid), jnp.float32) for _ in range(n_streams)]
        hb = [None] * n_streams
        for t in range(seq_len):
            for s in range(n_streams):
                r0 = t * b_pad + s * bs
                g = gate_ref[r0:r0 + bs, :]
                if t > 0:
                    g = jnp.dot(hb[s], whh, preferred_element_type=jnp.float32) + g

                ti = jnp.tanh(0.5 * g[:, 0 * hid:1 * hid])
                tf = jnp.tanh(0.5 * g[:, 1 * hid:2 * hid])
                tg = jnp.tanh(g[:, 2 * hid:3 * hid])
                to = jnp.tanh(0.5 * g[:, 3 * hid:4 * hid])

                c[s] = 0.5 * ((tf * c[s] + c[s]) + (ti * tg + tg))
                tc = jnp.tanh(c[s])
                h[s] = 0.5 * (to * tc + tc)
                hb[s] = h[s].astype(jnp.bfloat16)
                seq_ref[r0:r0 + bs, :] = hb[s]

        hid_ref[0] = jnp.concatenate(h, axis=0) if n_streams > 1 else h[0]
        cell_ref[0] = jnp.concatenate(c, axis=0) if n_streams > 1 else c[0]

    return body


def kernel(x, w_ih_all, w_hh_all, b_all):
    num_layers, d_pad, four_h = w_ih_all.shape
    hid = four_h // 4
    B, T, D = x.shape

    n_streams = 2
    b_pad = max(8 * n_streams, -(-B // (8 * n_streams)) * (8 * n_streams))
    if b_pad != B or d_pad != D:
        x = jnp.pad(x, ((0, b_pad - B), (0, 0), (0, d_pad - D)))

    body = _make_lstm_body(T, b_pad, d_pad, hid, n_streams)

    hidden, cell = pl.pallas_call(
        body,
        grid=(num_layers,),
        in_specs=[
            pl.BlockSpec((b_pad, T, d_pad), lambda l: (0, 0, 0)),
            pl.BlockSpec((1, d_pad, four_h), lambda l: (l, 0, 0)),
            pl.BlockSpec((1, hid, four_h), lambda l: (l, 0, 0)),
            pl.BlockSpec((1, 1, four_h), lambda l: (l, 0, 0)),
        ],
        out_specs=[
            pl.BlockSpec((1, b_pad, hid), lambda l: (l, 0, 0)),
            pl.BlockSpec((1, b_pad, hid), lambda l: (l, 0, 0)),
        ],
        out_shape=(
            jax.ShapeDtypeStruct((num_layers, b_pad, hid), jnp.float32),
            jax.ShapeDtypeStruct((num_layers, b_pad, hid), jnp.float32),
        ),
        scratch_shapes=[
            pltpu.VMEM((T * b_pad, d_pad), jnp.bfloat16),
            pltpu.VMEM((T * b_pad, hid), jnp.bfloat16),
            pltpu.VMEM((T * b_pad, four_h), jnp.float32),
        ],
        compiler_params=pltpu.CompilerParams(
            dimension_semantics=("arbitrary",)),
    )(x, w_ih_all, w_hh_all, b_all)

    if b_pad != B:
        hidden, cell = hidden[:, :B, :], cell[:, :B, :]
    return hidden, cell


# speedup vs baseline: 1.7880x; 1.1549x over previous
import jax
import jax.numpy as jnp
from jax.experimental import pallas as pl
from jax.experimental.pallas import tpu as pltpu


def _make_lstm_body(seq_len, b_pad, d_pad, hid, n_streams):
    bs = b_pad // n_streams

    def body(x_ref, wih_ref, whh_ref, b_ref,
             hid_ref, cell_ref,
             seq_ref, wcat_ref):
        wcat_ref[:hid, :] = whh_ref[0].astype(jnp.bfloat16)
        wcat_ref[hid:, :] = wih_ref[0].astype(jnp.bfloat16)
        b = b_ref[0]

        @pl.when(pl.program_id(0) == 0)
        def _():
            for t in range(seq_len):
                seq_ref[t * b_pad:(t + 1) * b_pad, :] = (
                    x_ref[:, t, :].astype(jnp.bfloat16))

        h = [jnp.zeros((bs, hid), jnp.float32) for _ in range(n_streams)]
        c = [jnp.zeros((bs, hid), jnp.float32) for _ in range(n_streams)]
        hb = [None] * n_streams
        for t in range(seq_len):
            for s in range(n_streams):
                r0 = t * b_pad + s * bs
                s_t = seq_ref[r0:r0 + bs, :]
                if t == 0:
                    g = jnp.dot(s_t, wcat_ref[hid:, :],
                                preferred_element_type=jnp.float32) + b
                else:
                    lhs = jnp.concatenate([hb[s], s_t], axis=1)
                    g = jnp.dot(lhs, wcat_ref[...],
                                preferred_element_type=jnp.float32) + b

                ti = jnp.tanh(0.5 * g[:, 0 * hid:1 * hid])
                tf = jnp.tanh(0.5 * g[:, 1 * hid:2 * hid])
                tg = jnp.tanh(g[:, 2 * hid:3 * hid])
                to = jnp.tanh(0.5 * g[:, 3 * hid:4 * hid])

                c[s] = 0.5 * ((tf * c[s] + c[s]) + (ti * tg + tg))
                tc = jnp.tanh(c[s])
                h[s] = 0.5 * (to * tc + tc)
                hb[s] = h[s].astype(jnp.bfloat16)
                seq_ref[r0:r0 + bs, :hid] = hb[s]

        hid_ref[0] = jnp.concatenate(h, axis=0) if n_streams > 1 else h[0]
        cell_ref[0] = jnp.concatenate(c, axis=0) if n_streams > 1 else c[0]

    return body


def kernel(x, w_ih_all, w_hh_all, b_all):
    num_layers, d_pad, four_h = w_ih_all.shape
    hid = four_h // 4
    B, T, D = x.shape

    n_streams = 2
    b_pad = max(8 * n_streams, -(-B // (8 * n_streams)) * (8 * n_streams))
    if b_pad != B or d_pad != D:
        x = jnp.pad(x, ((0, b_pad - B), (0, 0), (0, d_pad - D)))

    body = _make_lstm_body(T, b_pad, d_pad, hid, n_streams)

    hidden, cell = pl.pallas_call(
        body,
        grid=(num_layers,),
        in_specs=[
            pl.BlockSpec((b_pad, T, d_pad), lambda l: (0, 0, 0)),
            pl.BlockSpec((1, d_pad, four_h), lambda l: (l, 0, 0)),
            pl.BlockSpec((1, hid, four_h), lambda l: (l, 0, 0)),
            pl.BlockSpec((1, 1, four_h), lambda l: (l, 0, 0)),
        ],
        out_specs=[
            pl.BlockSpec((1, b_pad, hid), lambda l: (l, 0, 0)),
            pl.BlockSpec((1, b_pad, hid), lambda l: (l, 0, 0)),
        ],
        out_shape=(
            jax.ShapeDtypeStruct((num_layers, b_pad, hid), jnp.float32),
            jax.ShapeDtypeStruct((num_layers, b_pad, hid), jnp.float32),
        ),
        scratch_shapes=[
            pltpu.VMEM((T * b_pad, d_pad), jnp.bfloat16),
            pltpu.VMEM((hid + d_pad, four_h), jnp.bfloat16),
        ],
        compiler_params=pltpu.CompilerParams(
            dimension_semantics=("arbitrary",)),
    )(x, w_ih_all, w_hh_all, b_all)

    if b_pad != B:
        hidden, cell = hidden[:, :B, :], cell[:, :B, :]
    return hidden, cell
